# Initial kernel scaffold; baseline (speedup 1.0000x reference)
import jax
import jax.numpy as jnp
from jax.experimental import pallas as pl


def kernel(input_ids, attention_mask, embed_table, dense_W, dense_b, out_W, out_b, router_W, router_b, exp_W1, exp_b1, exp_ln_g, exp_ln_b, exp_W2, exp_b2, proj_W, proj_b, fc1_W, fc1_b, ln2_g, ln2_b, fc2_W, fc2_b):
    raise NotImplementedError("write your pallas kernel here")


# speedup vs baseline: 1.5475x; 1.5475x over previous
import functools

import jax
import jax.numpy as jnp
from jax import lax
from jax.experimental import pallas as pl
from jax.experimental.pallas import tpu as pltpu
from jax.experimental.pallas import tpu_sc as plsc

_B, _S, _H, _E, _K, _V, _D, _C = 32, 2048, 1024, 16, 4, 30522, 256, 3
_VBLK = 128
_NBLK = (_V + _VBLK - 1) // _VBLK
_VPAD = _NBLK * _VBLK
_NC, _NS, _L = 2, 16, 16


def _sc_counts_cls(ids, embed_table):
    mesh = plsc.VectorSubcoreMesh(core_axis_name="c", subcore_axis_name="s")

    @functools.partial(
        pl.kernel,
        mesh=mesh,
        compiler_params=pltpu.CompilerParams(needs_layout_passes=False),
        out_type=(
            jax.ShapeDtypeStruct((_B, _VPAD), jnp.float32),
            jax.ShapeDtypeStruct((_B, _H), jnp.float32),
        ),
        scratch_types=[
            pltpu.VMEM((_S,), jnp.int32),
            pltpu.VMEM((_VPAD,), jnp.float32),
            pltpu.VMEM((1, _H), jnp.float32),
            pltpu.SemaphoreType.DMA,
        ],
    )
    def k(ids_hbm, table_hbm, counts_hbm, cls_hbm, ids_v, counts_v, cls_v, sem):
        w = lax.axis_index("s") * _NC + lax.axis_index("c")
        pltpu.sync_copy(ids_hbm.at[w], ids_v)
        cls_dma = pltpu.async_copy(
            table_hbm.at[ids_v.at[pl.ds(0, 1)]], cls_v, sem
        )

        zero = jnp.zeros((_L,), jnp.float32)

        def zbody(i, carry):
            counts_v[pl.ds(i * _L, _L)] = zero
            return carry

        lax.fori_loop(0, _VPAD // _L, zbody, 0)

        ones = jnp.ones((_L,), jnp.float32)
        lanes = lax.iota(jnp.int32, _L)

        def body(i, carry):
            idx = ids_v[pl.ds(i * _L, _L)]
            for l in range(_L):
                plsc.addupdate_scatter(counts_v, [idx], ones, mask=lanes == l)
            return carry

        lax.fori_loop(0, _S // _L, body, 0)
        pltpu.sync_copy(counts_v, counts_hbm.at[w])
        cls_dma.wait()
        pltpu.sync_copy(cls_v, cls_hbm.at[pl.ds(w, 1)])

    return k(ids, embed_table)


def _mm_body(counts_ref, table_ref, acc_ref):
    j = pl.program_id(0)

    @pl.when(j == 0)
    def _():
        acc_ref[...] = jnp.zeros_like(acc_ref)

    t = table_ref[...]
    rows = lax.broadcasted_iota(jnp.int32, t.shape, 0) + j * _VBLK
    t = jnp.where(rows < _V, t, 0.0)
    acc_ref[...] += jnp.dot(
        counts_ref[...], t,
        preferred_element_type=jnp.float32,
        precision=lax.Precision.HIGHEST,
    )


def _counts_matmul(counts, embed_table):
    return pl.pallas_call(
        _mm_body,
        grid=(_NBLK,),
        in_specs=[
            pl.BlockSpec((_B, _VBLK), lambda j: (0, j)),
            pl.BlockSpec((_VBLK, _H), lambda j: (j, 0)),
        ],
        out_specs=pl.BlockSpec((_B, _H), lambda j: (0, 0)),
        out_shape=jax.ShapeDtypeStruct((_B, _H), jnp.float32),
    )(counts, embed_table)


def _dot(a, b):
    return jnp.dot(a, b, preferred_element_type=jnp.float32)


def _bf(x):
    return x.astype(jnp.bfloat16).astype(jnp.float32)


def _tiny_dot(a, b_ref_slice):
    kdim = a.shape[1]
    a = _bf(a)
    b = _bf(b_ref_slice)
    out = a[:, 0:1] * b[0:1, :]
    for i in range(1, kdim):
        out = out + a[:, i:i + 1] * b[i:i + 1, :]
    return out


def _epi_body(sum_ref, cls_ref, dW, db, oW, ob, rW, rb, W1, b1, lng, lnb,
              W2, b2, pW, pb, f1W, f1b, l2g, l2b, f2W, f2b, out_ref):
    cls = cls_ref[...]
    x = jnp.tanh(_dot(cls, dW[...]) + db[...])
    orig = _dot(x, oW[...]) + ob[...]

    rl = _dot(cls, rW[...]) + rb[...]
    colidx = lax.broadcasted_iota(jnp.int32, rl.shape, 1)
    work = rl
    selected = jnp.zeros(rl.shape, jnp.bool_)
    for _ in range(_K):
        mx = jnp.max(work, axis=-1, keepdims=True)
        is_mx = work == mx
        first_idx = jnp.min(jnp.where(is_mx, colidx, _E), axis=-1,
                            keepdims=True)
        first = colidx == first_idx
        selected = jnp.logical_or(selected, first)
        work = jnp.where(first, -1e30, work)
    m = jnp.max(rl, axis=-1, keepdims=True)
    ex = jnp.where(selected, jnp.exp(rl - m), 0.0)
    w_rt = ex / jnp.sum(ex, axis=-1, keepdims=True)

    mean = sum_ref[...] * (1.0 / _S)
    moe = jnp.zeros((_B, _C), jnp.float32)
    inv_sqrt2 = 0.7071067811865476
    for e in range(_E):
        h1 = _dot(mean, W1[e]) + b1[e:e + 1, :]
        mu = jnp.mean(h1, axis=-1, keepdims=True)
        var = jnp.mean((h1 - mu) ** 2, axis=-1, keepdims=True)
        h1 = (h1 - mu) / jnp.sqrt(var + 1e-5) * lng[e:e + 1, :] + lnb[e:e + 1, :]
        h1 = 0.5 * h1 * (1.0 + lax.erf(h1 * inv_sqrt2))
        h2 = _dot(h1, W2[e]) + b2[e:e + 1, :]
        el = _dot(h2, pW[...]) + pb[...]
        moe = moe + w_rt[:, e:e + 1] * el

    y = _tiny_dot(orig, f1W[0:_C, :]) + _tiny_dot(moe, f1W[_C:2 * _C, :])
    y = y + f1b[...]
    mu = jnp.mean(y, axis=-1, keepdims=True)
    var = jnp.mean((y - mu) ** 2, axis=-1, keepdims=True)
    y = (y - mu) / jnp.sqrt(var + 1e-5) * l2g[...] + l2b[...]
    y = jnp.maximum(y, 0.0)
    out_ref[...] = _tiny_dot(y, f2W[...]) + f2b[...]


def _epilogue(sum_emb, cls, dense_W, dense_b, out_W, out_b, router_W,
              router_b, exp_W1, exp_b1, exp_ln_g, exp_ln_b, exp_W2, exp_b2,
              proj_W, proj_b, fc1_W, fc1_b, ln2_g, ln2_b, fc2_W, fc2_b):
    args = (sum_emb, cls, dense_W, dense_b, out_W, out_b, router_W, router_b,
            exp_W1, exp_b1, exp_ln_g, exp_ln_b, exp_W2, exp_b2, proj_W,
            proj_b, fc1_W, fc1_b, ln2_g, ln2_b, fc2_W, fc2_b)
    return pl.pallas_call(
        _epi_body,
        out_shape=jax.ShapeDtypeStruct((_B, _C), jnp.float32),
    )(*args)


def kernel(input_ids, attention_mask, embed_table, dense_W, dense_b, out_W,
           out_b, router_W, router_b, exp_W1, exp_b1, exp_ln_g, exp_ln_b,
           exp_W2, exp_b2, proj_W, proj_b, fc1_W, fc1_b, ln2_g, ln2_b,
           fc2_W, fc2_b):
    ids = input_ids.astype(jnp.int32)
    counts, cls = _sc_counts_cls(ids, embed_table)
    sum_emb = _counts_matmul(counts, embed_table)
    return _epilogue(sum_emb, cls, dense_W, dense_b, out_W, out_b, router_W,
                     router_b, exp_W1, exp_b1, exp_ln_g, exp_ln_b, exp_W2,
                     exp_b2, proj_W, proj_b, fc1_W, fc1_b, ln2_g, ln2_b,
                     fc2_W, fc2_b)


# speedup vs baseline: 2.5580x; 1.6530x over previous
import functools

import jax
import jax.numpy as jnp
from jax import lax
from jax.experimental import pallas as pl
from jax.experimental.pallas import tpu as pltpu
from jax.experimental.pallas import tpu_sc as plsc

_B, _S, _H, _E, _K, _V, _D, _C = 32, 2048, 1024, 16, 4, 30522, 256, 3
_VBLK = 512
_NBLK = (_V + _VBLK - 1) // _VBLK
_VPAD = _NBLK * _VBLK
_NC, _NS, _L = 2, 16, 16


def _sc_counts_cls(ids, embed_table):
    mesh = plsc.VectorSubcoreMesh(core_axis_name="c", subcore_axis_name="s")

    @functools.partial(
        pl.kernel,
        mesh=mesh,
        compiler_params=pltpu.CompilerParams(needs_layout_passes=False),
        out_type=(
            jax.ShapeDtypeStruct((_B, _VPAD), jnp.float32),
            jax.ShapeDtypeStruct((_B, _H), jnp.float32),
        ),
        scratch_types=[
            pltpu.VMEM((_S,), jnp.int32),
            pltpu.VMEM((_VPAD,), jnp.float32),
            pltpu.VMEM((1, _H), jnp.float32),
            pltpu.SemaphoreType.DMA,
        ],
    )
    def k(ids_hbm, table_hbm, counts_hbm, cls_hbm, ids_v, counts_v, cls_v, sem):
        w = lax.axis_index("s") * _NC + lax.axis_index("c")
        pltpu.sync_copy(ids_hbm.at[w], ids_v)
        cls_dma = pltpu.async_copy(
            table_hbm.at[ids_v.at[pl.ds(0, 1)]], cls_v, sem
        )

        zero = jnp.zeros((_L,), jnp.float32)

        def zbody(i, carry):
            counts_v[pl.ds(i * _L, _L)] = zero
            return carry

        lax.fori_loop(0, _VPAD // _L, zbody, 0)

        ones = jnp.ones((_L,), jnp.float32)
        lanes = lax.iota(jnp.int32, _L)

        def body(i, carry):
            idx = ids_v[pl.ds(i * _L, _L)]
            for l in range(_L):
                plsc.addupdate_scatter(counts_v, [idx], ones, mask=lanes == l)
            return carry

        lax.fori_loop(0, _S // _L, body, 0)
        pltpu.sync_copy(counts_v, counts_hbm.at[w])
        cls_dma.wait()
        pltpu.sync_copy(cls_v, cls_hbm.at[pl.ds(w, 1)])

    return k(ids, embed_table)


def _dot(a, b):
    return jnp.dot(a, b, preferred_element_type=jnp.float32)


def _bf(x):
    return x.astype(jnp.bfloat16).astype(jnp.float32)


def _tiny_dot(a, b_ref_slice):
    kdim = a.shape[1]
    a = _bf(a)
    b = _bf(b_ref_slice)
    out = a[:, 0:1] * b[0:1, :]
    for i in range(1, kdim):
        out = out + a[:, i:i + 1] * b[i:i + 1, :]
    return out


def _fused_body(counts_ref, table_ref, cls_ref, dW, db, oW, ob, rW, rb, W1,
                b1, lng, lnb, W2, b2, pW, pb, f1W, f1b, l2g, l2b, f2W, f2b,
                out_ref, acc_ref):
    j = pl.program_id(0)

    @pl.when(j == 0)
    def _():
        acc_ref[...] = jnp.zeros_like(acc_ref)

    t = table_ref[...]
    rows = lax.broadcasted_iota(jnp.int32, t.shape, 0) + j * _VBLK
    t = jnp.where(rows < _V, t, 0.0)
    acc_ref[...] += jnp.dot(
        counts_ref[...], t,
        preferred_element_type=jnp.float32,
        precision=lax.Precision.HIGHEST,
    )

    @pl.when(j == _NBLK - 1)
    def _():
        cls = cls_ref[...]
        x = jnp.tanh(_dot(cls, dW[...]) + db[...])
        orig = _dot(x, oW[...]) + ob[...]

        rl = _dot(cls, rW[...]) + rb[...]
        colidx = lax.broadcasted_iota(jnp.int32, rl.shape, 1)
        work = rl
        selected = jnp.zeros(rl.shape, jnp.bool_)
        for _unused in range(_K):
            mx = jnp.max(work, axis=-1, keepdims=True)
            is_mx = work == mx
            first_idx = jnp.min(jnp.where(is_mx, colidx, _E), axis=-1,
                                keepdims=True)
            first = colidx == first_idx
            selected = jnp.logical_or(selected, first)
            work = jnp.where(first, -1e30, work)
        m = jnp.max(rl, axis=-1, keepdims=True)
        ex = jnp.where(selected, jnp.exp(rl - m), 0.0)
        w_rt = ex / jnp.sum(ex, axis=-1, keepdims=True)

        mean = acc_ref[...] * (1.0 / _S)
        moe = jnp.zeros((_B, _C), jnp.float32)
        inv_sqrt2 = 0.7071067811865476
        for e in range(_E):
            h1 = _dot(mean, W1[e]) + b1[e:e + 1, :]
            mu = jnp.mean(h1, axis=-1, keepdims=True)
            var = jnp.mean((h1 - mu) ** 2, axis=-1, keepdims=True)
            h1 = ((h1 - mu) / jnp.sqrt(var + 1e-5) * lng[e:e + 1, :]
                  + lnb[e:e + 1, :])
            h1 = 0.5 * h1 * (1.0 + lax.erf(h1 * inv_sqrt2))
            h2 = _dot(h1, W2[e]) + b2[e:e + 1, :]
            el = _dot(h2, pW[...]) + pb[...]
            moe = moe + w_rt[:, e:e + 1] * el

        y = _tiny_dot(orig, f1W[0:_C, :]) + _tiny_dot(moe, f1W[_C:2 * _C, :])
        y = y + f1b[...]
        mu = jnp.mean(y, axis=-1, keepdims=True)
        var = jnp.mean((y - mu) ** 2, axis=-1, keepdims=True)
        y = (y - mu) / jnp.sqrt(var + 1e-5) * l2g[...] + l2b[...]
        y = jnp.maximum(y, 0.0)
        out_ref[...] = _tiny_dot(y, f2W[...]) + f2b[...]


def _fused_tc(counts, embed_table, cls, dense_W, dense_b, out_W, out_b,
              router_W, router_b, exp_W1, exp_b1, exp_ln_g, exp_ln_b, exp_W2,
              exp_b2, proj_W, proj_b, fc1_W, fc1_b, ln2_g, ln2_b, fc2_W,
              fc2_b):
    wargs = (cls, dense_W, dense_b, out_W, out_b, router_W, router_b, exp_W1,
             exp_b1, exp_ln_g, exp_ln_b, exp_W2, exp_b2, proj_W, proj_b,
             fc1_W, fc1_b, ln2_g, ln2_b, fc2_W, fc2_b)

    def const_spec(a):
        nd = a.ndim
        return pl.BlockSpec(a.shape, lambda j, _n=nd: (0,) * _n)

    return pl.pallas_call(
        _fused_body,
        grid=(_NBLK,),
        in_specs=[
            pl.BlockSpec((_B, _VBLK), lambda j: (0, j)),
            pl.BlockSpec((_VBLK, _H), lambda j: (j, 0)),
        ] + [const_spec(a) for a in wargs],
        out_specs=pl.BlockSpec((_B, _C), lambda j: (0, 0)),
        out_shape=jax.ShapeDtypeStruct((_B, _C), jnp.float32),
        scratch_shapes=[pltpu.VMEM((_B, _H), jnp.float32)],
    )(counts, embed_table, *wargs)


def kernel(input_ids, attention_mask, embed_table, dense_W, dense_b, out_W,
           out_b, router_W, router_b, exp_W1, exp_b1, exp_ln_g, exp_ln_b,
           exp_W2, exp_b2, proj_W, proj_b, fc1_W, fc1_b, ln2_g, ln2_b,
           fc2_W, fc2_b):
    ids = input_ids.astype(jnp.int32)
    counts, cls = _sc_counts_cls(ids, embed_table)
    return _fused_tc(counts, embed_table, cls, dense_W, dense_b, out_W,
                     out_b, router_W, router_b, exp_W1, exp_b1, exp_ln_g,
                     exp_ln_b, exp_W2, exp_b2, proj_W, proj_b, fc1_W, fc1_b,
                     ln2_g, ln2_b, fc2_W, fc2_b)


# speedup vs baseline: 2.8335x; 1.1077x over previous
import functools

import jax
import jax.numpy as jnp
from jax import lax
from jax.experimental import pallas as pl
from jax.experimental.pallas import tpu as pltpu
from jax.experimental.pallas import tpu_sc as plsc

_B, _S, _H, _E, _K, _V, _D, _C = 32, 2048, 1024, 16, 4, 30522, 256, 3
_VBLK = 1024
_NBLK = (_V + _VBLK - 1) // _VBLK
_VPAD = _NBLK * _VBLK
_NC, _NS, _L = 2, 16, 16


def _sc_counts_cls(ids, embed_table):
    mesh = plsc.VectorSubcoreMesh(core_axis_name="c", subcore_axis_name="s")

    @functools.partial(
        pl.kernel,
        mesh=mesh,
        compiler_params=pltpu.CompilerParams(needs_layout_passes=False),
        out_type=(
            jax.ShapeDtypeStruct((_B, _VPAD), jnp.float32),
            jax.ShapeDtypeStruct((_B, _H), jnp.float32),
        ),
        scratch_types=[
            pltpu.VMEM((_S,), jnp.int32),
            pltpu.VMEM((_VPAD,), jnp.float32),
            pltpu.VMEM((1, _H), jnp.float32),
            pltpu.SemaphoreType.DMA,
        ],
    )
    def k(ids_hbm, table_hbm, counts_hbm, cls_hbm, ids_v, counts_v, cls_v, sem):
        w = lax.axis_index("s") * _NC + lax.axis_index("c")
        pltpu.sync_copy(ids_hbm.at[w], ids_v)
        cls_dma = pltpu.async_copy(
            table_hbm.at[ids_v.at[pl.ds(0, 1)]], cls_v, sem
        )

        zero = jnp.zeros((_L,), jnp.float32)

        def zbody(i, carry):
            counts_v[pl.ds(i * _L, _L)] = zero
            return carry

        lax.fori_loop(0, _VPAD // _L, zbody, 0)

        ones = jnp.ones((_L,), jnp.float32)
        lanes = lax.iota(jnp.int32, _L)

        def body(i, carry):
            idx = ids_v[pl.ds(i * _L, _L)]
            for l in range(_L):
                plsc.addupdate_scatter(counts_v, [idx], ones, mask=lanes == l)
            return carry

        lax.fori_loop(0, _S // _L, body, 0)
        pltpu.sync_copy(counts_v, counts_hbm.at[w])
        cls_dma.wait()
        pltpu.sync_copy(cls_v, cls_hbm.at[pl.ds(w, 1)])

    return k(ids, embed_table)


def _dot(a, b):
    return jnp.dot(a, b, preferred_element_type=jnp.float32)


def _bf(x):
    return x.astype(jnp.bfloat16).astype(jnp.float32)


def _tiny_dot(a, b_ref_slice):
    kdim = a.shape[1]
    a = _bf(a)
    b = _bf(b_ref_slice)
    out = a[:, 0:1] * b[0:1, :]
    for i in range(1, kdim):
        out = out + a[:, i:i + 1] * b[i:i + 1, :]
    return out


def _fused_body(counts_ref, table_ref, cls_ref, dW, db, oW, ob, rW, rb, W1,
                b1, lng, lnb, W2, b2, pW, pb, f1W, f1b, l2g, l2b, f2W, f2b,
                out_ref, acc_ref):
    j = pl.program_id(0)

    @pl.when(j == 0)
    def _():
        acc_ref[...] = jnp.zeros_like(acc_ref)

    t = table_ref[...]
    rows = lax.broadcasted_iota(jnp.int32, t.shape, 0) + j * _VBLK
    t = jnp.where(rows < _V, t, 0.0)
    acc_ref[...] += jnp.dot(
        counts_ref[...], t,
        preferred_element_type=jnp.float32,
        precision=lax.Precision.HIGHEST,
    )

    @pl.when(j == _NBLK - 1)
    def _():
        cls = cls_ref[...]
        x = jnp.tanh(_dot(cls, dW[...]) + db[...])
        orig = _dot(x, oW[...]) + ob[...]

        rl = _dot(cls, rW[...]) + rb[...]
        colidx = lax.broadcasted_iota(jnp.int32, rl.shape, 1)
        work = rl
        selected = jnp.zeros(rl.shape, jnp.bool_)
        for _unused in range(_K):
            mx = jnp.max(work, axis=-1, keepdims=True)
            is_mx = work == mx
            first_idx = jnp.min(jnp.where(is_mx, colidx, _E), axis=-1,
                                keepdims=True)
            first = colidx == first_idx
            selected = jnp.logical_or(selected, first)
            work = jnp.where(first, -1e30, work)
        m = jnp.max(rl, axis=-1, keepdims=True)
        ex = jnp.where(selected, jnp.exp(rl - m), 0.0)
        w_rt = ex / jnp.sum(ex, axis=-1, keepdims=True)

        mean = acc_ref[...] * (1.0 / _S)
        moe = jnp.zeros((_B, _C), jnp.float32)
        inv_sqrt2 = 0.7071067811865476
        for e in range(_E):
            h1 = _dot(mean, W1[e]) + b1[e:e + 1, :]
            mu = jnp.mean(h1, axis=-1, keepdims=True)
            var = jnp.mean((h1 - mu) ** 2, axis=-1, keepdims=True)
            h1 = ((h1 - mu) / jnp.sqrt(var + 1e-5) * lng[e:e + 1, :]
                  + lnb[e:e + 1, :])
            h1 = 0.5 * h1 * (1.0 + lax.erf(h1 * inv_sqrt2))
            h2 = _dot(h1, W2[e]) + b2[e:e + 1, :]
            el = _dot(h2, pW[...]) + pb[...]
            moe = moe + w_rt[:, e:e + 1] * el

        y = _tiny_dot(orig, f1W[0:_C, :]) + _tiny_dot(moe, f1W[_C:2 * _C, :])
        y = y + f1b[...]
        mu = jnp.mean(y, axis=-1, keepdims=True)
        var = jnp.mean((y - mu) ** 2, axis=-1, keepdims=True)
        y = (y - mu) / jnp.sqrt(var + 1e-5) * l2g[...] + l2b[...]
        y = jnp.maximum(y, 0.0)
        out_ref[...] = _tiny_dot(y, f2W[...]) + f2b[...]


def _fused_tc(counts, embed_table, cls, dense_W, dense_b, out_W, out_b,
              router_W, router_b, exp_W1, exp_b1, exp_ln_g, exp_ln_b, exp_W2,
              exp_b2, proj_W, proj_b, fc1_W, fc1_b, ln2_g, ln2_b, fc2_W,
              fc2_b):
    wargs = (cls, dense_W, dense_b, out_W, out_b, router_W, router_b, exp_W1,
             exp_b1, exp_ln_g, exp_ln_b, exp_W2, exp_b2, proj_W, proj_b,
             fc1_W, fc1_b, ln2_g, ln2_b, fc2_W, fc2_b)

    def const_spec(a):
        nd = a.ndim
        return pl.BlockSpec(a.shape, lambda j, _n=nd: (0,) * _n)

    return pl.pallas_call(
        _fused_body,
        grid=(_NBLK,),
        in_specs=[
            pl.BlockSpec((_B, _VBLK), lambda j: (0, j)),
            pl.BlockSpec((_VBLK, _H), lambda j: (j, 0)),
        ] + [const_spec(a) for a in wargs],
        out_specs=pl.BlockSpec((_B, _C), lambda j: (0, 0)),
        out_shape=jax.ShapeDtypeStruct((_B, _C), jnp.float32),
        scratch_shapes=[pltpu.VMEM((_B, _H), jnp.float32)],
    )(counts, embed_table, *wargs)


def kernel(input_ids, attention_mask, embed_table, dense_W, dense_b, out_W,
           out_b, router_W, router_b, exp_W1, exp_b1, exp_ln_g, exp_ln_b,
           exp_W2, exp_b2, proj_W, proj_b, fc1_W, fc1_b, ln2_g, ln2_b,
           fc2_W, fc2_b):
    ids = input_ids.astype(jnp.int32)
    counts, cls = _sc_counts_cls(ids, embed_table)
    return _fused_tc(counts, embed_table, cls, dense_W, dense_b, out_W,
                     out_b, router_W, router_b, exp_W1, exp_b1, exp_ln_g,
                     exp_ln_b, exp_W2, exp_b2, proj_W, proj_b, fc1_W, fc1_b,
                     ln2_g, ln2_b, fc2_W, fc2_b)


# speedup vs baseline: 2.8751x; 1.0147x over previous
import functools

import jax
import jax.numpy as jnp
from jax import lax
from jax.experimental import pallas as pl
from jax.experimental.pallas import tpu as pltpu
from jax.experimental.pallas import tpu_sc as plsc

_B, _S, _H, _E, _K, _V, _D, _C = 32, 2048, 1024, 16, 4, 30522, 256, 3
_VBLK = 2048
_NBLK = (_V + _VBLK - 1) // _VBLK
_VPAD = _NBLK * _VBLK
_NC, _NS, _L = 2, 16, 16


def _sc_counts_cls(ids, embed_table):
    mesh = plsc.VectorSubcoreMesh(core_axis_name="c", subcore_axis_name="s")

    @functools.partial(
        pl.kernel,
        mesh=mesh,
        compiler_params=pltpu.CompilerParams(needs_layout_passes=False),
        out_type=(
            jax.ShapeDtypeStruct((_B, _VPAD), jnp.float32),
            jax.ShapeDtypeStruct((_B, _H), jnp.float32),
        ),
        scratch_types=[
            pltpu.VMEM((_S,), jnp.int32),
            pltpu.VMEM((_VPAD,), jnp.float32),
            pltpu.VMEM((1, _H), jnp.float32),
            pltpu.SemaphoreType.DMA,
        ],
    )
    def k(ids_hbm, table_hbm, counts_hbm, cls_hbm, ids_v, counts_v, cls_v, sem):
        w = lax.axis_index("s") * _NC + lax.axis_index("c")
        pltpu.sync_copy(ids_hbm.at[w], ids_v)
        cls_dma = pltpu.async_copy(
            table_hbm.at[ids_v.at[pl.ds(0, 1)]], cls_v, sem
        )

        zero = jnp.zeros((_L,), jnp.float32)

        def zbody(i, carry):
            counts_v[pl.ds(i * _L, _L)] = zero
            return carry

        lax.fori_loop(0, _VPAD // _L, zbody, 0)

        ones = jnp.ones((_L,), jnp.float32)
        lanes = lax.iota(jnp.int32, _L)

        def body(i, carry):
            idx = ids_v[pl.ds(i * _L, _L)]
            for l in range(_L):
                plsc.addupdate_scatter(counts_v, [idx], ones, mask=lanes == l)
            return carry

        lax.fori_loop(0, _S // _L, body, 0)
        pltpu.sync_copy(counts_v, counts_hbm.at[w])
        cls_dma.wait()
        pltpu.sync_copy(cls_v, cls_hbm.at[pl.ds(w, 1)])

    return k(ids, embed_table)


def _dot(a, b):
    return jnp.dot(a, b, preferred_element_type=jnp.float32)


def _bf(x):
    return x.astype(jnp.bfloat16).astype(jnp.float32)


def _tiny_dot(a, b_ref_slice):
    kdim = a.shape[1]
    a = _bf(a)
    b = _bf(b_ref_slice)
    out = a[:, 0:1] * b[0:1, :]
    for i in range(1, kdim):
        out = out + a[:, i:i + 1] * b[i:i + 1, :]
    return out


def _fused_body(counts_ref, table_ref, cls_ref, dW, db, oW, ob, rW, rb, W1,
                b1, lng, lnb, W2, b2, pW, pb, f1W, f1b, l2g, l2b, f2W, f2b,
                out_ref, acc_ref):
    j = pl.program_id(0)

    @pl.when(j == 0)
    def _():
        acc_ref[...] = jnp.zeros_like(acc_ref)

    t = table_ref[...]
    rows = lax.broadcasted_iota(jnp.int32, t.shape, 0) + j * _VBLK
    t = jnp.where(rows < _V, t, 0.0)
    acc_ref[...] += jnp.dot(
        counts_ref[...], t,
        preferred_element_type=jnp.float32,
        precision=lax.Precision.HIGHEST,
    )

    @pl.when(j == _NBLK - 1)
    def _():
        cls = cls_ref[...]
        x = jnp.tanh(_dot(cls, dW[...]) + db[...])
        orig = _dot(x, oW[...]) + ob[...]

        rl = _dot(cls, rW[...]) + rb[...]
        colidx = lax.broadcasted_iota(jnp.int32, rl.shape, 1)
        work = rl
        selected = jnp.zeros(rl.shape, jnp.bool_)
        for _unused in range(_K):
            mx = jnp.max(work, axis=-1, keepdims=True)
            is_mx = work == mx
            first_idx = jnp.min(jnp.where(is_mx, colidx, _E), axis=-1,
                                keepdims=True)
            first = colidx == first_idx
            selected = jnp.logical_or(selected, first)
            work = jnp.where(first, -1e30, work)
        m = jnp.max(rl, axis=-1, keepdims=True)
        ex = jnp.where(selected, jnp.exp(rl - m), 0.0)
        w_rt = ex / jnp.sum(ex, axis=-1, keepdims=True)

        mean = acc_ref[...] * (1.0 / _S)
        moe = jnp.zeros((_B, _C), jnp.float32)
        inv_sqrt2 = 0.7071067811865476
        for e in range(_E):
            h1 = _dot(mean, W1[e]) + b1[e:e + 1, :]
            mu = jnp.mean(h1, axis=-1, keepdims=True)
            var = jnp.mean((h1 - mu) ** 2, axis=-1, keepdims=True)
            h1 = ((h1 - mu) / jnp.sqrt(var + 1e-5) * lng[e:e + 1, :]
                  + lnb[e:e + 1, :])
            h1 = 0.5 * h1 * (1.0 + lax.erf(h1 * inv_sqrt2))
            h2 = _dot(h1, W2[e]) + b2[e:e + 1, :]
            el = _dot(h2, pW[...]) + pb[...]
            moe = moe + w_rt[:, e:e + 1] * el

        y = _tiny_dot(orig, f1W[0:_C, :]) + _tiny_dot(moe, f1W[_C:2 * _C, :])
        y = y + f1b[...]
        mu = jnp.mean(y, axis=-1, keepdims=True)
        var = jnp.mean((y - mu) ** 2, axis=-1, keepdims=True)
        y = (y - mu) / jnp.sqrt(var + 1e-5) * l2g[...] + l2b[...]
        y = jnp.maximum(y, 0.0)
        out_ref[...] = _tiny_dot(y, f2W[...]) + f2b[...]


def _fused_tc(counts, embed_table, cls, dense_W, dense_b, out_W, out_b,
              router_W, router_b, exp_W1, exp_b1, exp_ln_g, exp_ln_b, exp_W2,
              exp_b2, proj_W, proj_b, fc1_W, fc1_b, ln2_g, ln2_b, fc2_W,
              fc2_b):
    wargs = (cls, dense_W, dense_b, out_W, out_b, router_W, router_b, exp_W1,
             exp_b1, exp_ln_g, exp_ln_b, exp_W2, exp_b2, proj_W, proj_b,
             fc1_W, fc1_b, ln2_g, ln2_b, fc2_W, fc2_b)

    def const_spec(a):
        nd = a.ndim
        return pl.BlockSpec(a.shape, lambda j, _n=nd: (0,) * _n)

    return pl.pallas_call(
        _fused_body,
        grid=(_NBLK,),
        in_specs=[
            pl.BlockSpec((_B, _VBLK), lambda j: (0, j)),
            pl.BlockSpec((_VBLK, _H), lambda j: (j, 0)),
        ] + [const_spec(a) for a in wargs],
        out_specs=pl.BlockSpec((_B, _C), lambda j: (0, 0)),
        out_shape=jax.ShapeDtypeStruct((_B, _C), jnp.float32),
        scratch_shapes=[pltpu.VMEM((_B, _H), jnp.float32)],
    )(counts, embed_table, *wargs)


def kernel(input_ids, attention_mask, embed_table, dense_W, dense_b, out_W,
           out_b, router_W, router_b, exp_W1, exp_b1, exp_ln_g, exp_ln_b,
           exp_W2, exp_b2, proj_W, proj_b, fc1_W, fc1_b, ln2_g, ln2_b,
           fc2_W, fc2_b):
    ids = input_ids.astype(jnp.int32)
    counts, cls = _sc_counts_cls(ids, embed_table)
    return _fused_tc(counts, embed_table, cls, dense_W, dense_b, out_W,
                     out_b, router_W, router_b, exp_W1, exp_b1, exp_ln_g,
                     exp_ln_b, exp_W2, exp_b2, proj_W, proj_b, fc1_W, fc1_b,
                     ln2_g, ln2_b, fc2_W, fc2_b)
